# baseline (speedup 1.0000x reference)
import jax
import jax.numpy as jnp
from jax.experimental import pallas as pl

_B = 4096
_K = 8192
_E = 32
_TB = 256
_TK = 2048


def _vq_onehot_kernel(x_ref, et_ref, out_ref):
    x = x_ref[:]
    x_sq = jnp.sum(x * x, axis=1, keepdims=True)
    nk = _K // _TK

    bmin = jnp.full((_TB, 1), jnp.inf, dtype=jnp.float32)
    bidx = jnp.zeros((_TB, 1), dtype=jnp.int32)
    for j in range(nk):
        et = et_ref[:, j * _TK:(j + 1) * _TK]
        s = jax.lax.dot_general(
            x, et, (((1,), (0,)), ((), ())), preferred_element_type=jnp.float32
        )
        e_sq = jnp.sum(et * et, axis=0, keepdims=True)
        d = (x_sq + e_sq) - 2.0 * s
        lmin = jnp.min(d, axis=1, keepdims=True)
        iota = jax.lax.broadcasted_iota(jnp.int32, (_TB, _TK), 1)
        lidx = jnp.min(
            jnp.where(d == lmin, iota, _K), axis=1, keepdims=True
        ) + j * _TK
        upd = lmin < bmin
        bmin = jnp.where(upd, lmin, bmin)
        bidx = jnp.where(upd, lidx, bidx)

    for j in range(nk):
        iota = jax.lax.broadcasted_iota(jnp.int32, (_TB, _TK), 1) + j * _TK
        out_ref[:, j * _TK:(j + 1) * _TK] = (iota == bidx).astype(jnp.float32)


def kernel(inputs, embeddings_weight):
    b = inputs.shape[0]
    flat = inputs.reshape(b, _E)
    et = embeddings_weight.reshape(_K, _E).T
    return pl.pallas_call(
        _vq_onehot_kernel,
        grid=(b // _TB,),
        in_specs=[
            pl.BlockSpec((_TB, _E), lambda i: (i, 0)),
            pl.BlockSpec((_E, _K), lambda i: (0, 0)),
        ],
        out_specs=pl.BlockSpec((_TB, _K), lambda i: (i, 0)),
        out_shape=jax.ShapeDtypeStruct((b, _K), jnp.float32),
    )(flat, et)


# speedup vs baseline: 1.1526x; 1.1526x over previous
import jax
import jax.numpy as jnp
from jax.experimental import pallas as pl

_B = 4096
_K = 8192
_E = 32
_TB = 256
_TK = 2048


def _vq_onehot_kernel(x_ref, et_ref, iota_ref, out_ref):
    x = x_ref[:]
    x_sq = jnp.sum(x * x, axis=1, keepdims=True)
    xm2 = x * -2.0
    iota = iota_ref[:]
    nk = _K // _TK

    bmin = jnp.full((_TB, 1), jnp.inf, dtype=jnp.float32)
    bidx = jnp.zeros((_TB, 1), dtype=jnp.float32)
    for j in range(nk):
        et = et_ref[:, j * _TK:(j + 1) * _TK]
        s = jax.lax.dot_general(
            xm2, et, (((1,), (0,)), ((), ())), preferred_element_type=jnp.float32
        )
        e_sq = jnp.sum(et * et, axis=0, keepdims=True)
        d = (x_sq + e_sq) + s
        lmin = jnp.min(d, axis=1, keepdims=True)
        lidx = jnp.min(
            jnp.where(d == lmin, iota, float(_K)), axis=1, keepdims=True
        ) + float(j * _TK)
        upd = lmin < bmin
        bmin = jnp.where(upd, lmin, bmin)
        bidx = jnp.where(upd, lidx, bidx)

    for j in range(nk):
        bj = bidx - float(j * _TK)
        out_ref[:, j * _TK:(j + 1) * _TK] = (iota == bj).astype(jnp.float32)


def kernel(inputs, embeddings_weight):
    b = inputs.shape[0]
    flat = inputs.reshape(b, _E)
    et = embeddings_weight.reshape(_K, _E).T
    iota = jnp.arange(_TK, dtype=jnp.float32)[None, :]
    return pl.pallas_call(
        _vq_onehot_kernel,
        grid=(b // _TB,),
        in_specs=[
            pl.BlockSpec((_TB, _E), lambda i: (i, 0)),
            pl.BlockSpec((_E, _K), lambda i: (0, 0)),
            pl.BlockSpec((1, _TK), lambda i: (0, 0)),
        ],
        out_specs=pl.BlockSpec((_TB, _K), lambda i: (i, 0)),
        out_shape=jax.ShapeDtypeStruct((b, _K), jnp.float32),
    )(flat, et, iota)


# speedup vs baseline: 1.1663x; 1.0119x over previous
import jax
import jax.numpy as jnp
from jax.experimental import pallas as pl

_B = 4096
_K = 8192
_E = 32
_TB = 512
_TK = 1024


def _vq_onehot_kernel(x_ref, et_ref, iota_ref, out_ref):
    x = x_ref[:]
    x_sq = jnp.sum(x * x, axis=1, keepdims=True)
    xm2 = x * -2.0
    iota = iota_ref[:]
    nk = _K // _TK

    bmin = jnp.full((_TB, 1), jnp.inf, dtype=jnp.float32)
    bidx = jnp.zeros((_TB, 1), dtype=jnp.float32)
    for j in range(nk):
        et = et_ref[:, j * _TK:(j + 1) * _TK]
        s = jax.lax.dot_general(
            xm2, et, (((1,), (0,)), ((), ())), preferred_element_type=jnp.float32
        )
        e_sq = jnp.sum(et * et, axis=0, keepdims=True)
        d = (x_sq + e_sq) + s
        lmin = jnp.min(d, axis=1, keepdims=True)
        lidx = jnp.min(
            jnp.where(d == lmin, iota, float(_K)), axis=1, keepdims=True
        ) + float(j * _TK)
        upd = lmin < bmin
        bmin = jnp.where(upd, lmin, bmin)
        bidx = jnp.where(upd, lidx, bidx)

    for j in range(nk):
        bj = bidx - float(j * _TK)
        out_ref[:, j * _TK:(j + 1) * _TK] = (iota == bj).astype(jnp.float32)


def kernel(inputs, embeddings_weight):
    b = inputs.shape[0]
    flat = inputs.reshape(b, _E)
    et = embeddings_weight.reshape(_K, _E).T
    iota = jnp.arange(_TK, dtype=jnp.float32)[None, :]
    return pl.pallas_call(
        _vq_onehot_kernel,
        grid=(b // _TB,),
        in_specs=[
            pl.BlockSpec((_TB, _E), lambda i: (i, 0)),
            pl.BlockSpec((_E, _K), lambda i: (0, 0)),
            pl.BlockSpec((1, _TK), lambda i: (0, 0)),
        ],
        out_specs=pl.BlockSpec((_TB, _K), lambda i: (i, 0)),
        out_shape=jax.ShapeDtypeStruct((b, _K), jnp.float32),
    )(flat, et, iota)
